# speedup vs baseline: 7.3799x; 7.3799x over previous
import functools

import jax
import jax.numpy as jnp
from jax import lax
from jax.experimental import pallas as pl
from jax.experimental.pallas import tpu as pltpu
from jax.experimental.pallas import tpu_sc as plsc

D = 128
NW = 32
GATHER = 128
CHUNK = 256
IDX_ROWS = CHUNK // GATHER


def _build(total_rows: int):
    per_w = total_rows // NW
    iters = per_w // CHUNK
    mesh = plsc.VectorSubcoreMesh(core_axis_name="c", subcore_axis_name="s")

    @functools.partial(
        pl.kernel,
        mesh=mesh,
        out_type=jax.ShapeDtypeStruct((total_rows, D), jnp.float32),
        scratch_types=[
            pltpu.VMEM((IDX_ROWS, GATHER), jnp.int32),
            pltpu.VMEM((CHUNK, D), jnp.float32),
            pltpu.SemaphoreType.DMA,
        ],
    )
    def gather_kernel(table_hbm, idx_hbm, out_hbm, idx_v, rows_v, sem):
        wid = lax.axis_index("s") * 2 + lax.axis_index("c")
        base_irow = wid * (per_w // GATHER)

        def body(g, carry):
            irow = base_irow + g * IDX_ROWS
            pltpu.sync_copy(idx_hbm.at[pl.ds(irow, IDX_ROWS)], idx_v)
            cps = [
                pltpu.async_copy(
                    table_hbm.at[idx_v.at[j]],
                    rows_v.at[pl.ds(j * GATHER, GATHER)],
                    sem,
                )
                for j in range(IDX_ROWS)
            ]
            for cp in cps:
                cp.wait()
            pltpu.sync_copy(rows_v, out_hbm.at[pl.ds(irow * GATHER, CHUNK)])
            return carry

        lax.fori_loop(0, iters, body, 0)

    return gather_kernel


def kernel(x, embeddings):
    b, t = x.shape
    total = b * t
    idx2d = x.astype(jnp.int32).reshape(total // GATHER, GATHER)
    out = _build(total)(embeddings, idx2d)
    return out.reshape(b, t, embeddings.shape[1])


# speedup vs baseline: 9.7631x; 1.3229x over previous
import functools

import jax
import jax.numpy as jnp
from jax import lax
from jax.experimental import pallas as pl
from jax.experimental.pallas import tpu as pltpu
from jax.experimental.pallas import tpu_sc as plsc

D = 128
NW = 32
GATHER = 128
IDX_ROWS = 2
CHUNK = IDX_ROWS * GATHER


def _build(total_rows: int):
    per_w = total_rows // NW
    n_chunks = per_w // CHUNK
    n_outer = n_chunks // 2
    mesh = plsc.VectorSubcoreMesh(core_axis_name="c", subcore_axis_name="s")

    @functools.partial(
        pl.kernel,
        mesh=mesh,
        out_type=jax.ShapeDtypeStruct((total_rows, D), jnp.float32),
        scratch_types=[
            pltpu.VMEM((2, IDX_ROWS, GATHER), jnp.int32),
            pltpu.VMEM((2, CHUNK, D), jnp.float32),
            pltpu.SemaphoreType.DMA,
            pltpu.SemaphoreType.DMA,
            pltpu.SemaphoreType.DMA,
            pltpu.SemaphoreType.DMA,
        ],
    )
    def gather_kernel(table_hbm, idx_hbm, out_hbm, idx_v, rows_v, isem, gsem, os0, os1):
        wid = lax.axis_index("s") * 2 + lax.axis_index("c")
        base_irow = wid * (per_w // GATHER)
        osems = (os0, os1)

        def idx_rows_of(c):
            return pl.ds(base_irow + c * IDX_ROWS, IDX_ROWS)

        def out_rows_of(c):
            return pl.ds((base_irow + c * IDX_ROWS) * GATHER, CHUNK)

        def fire_gathers(b):
            for j in range(IDX_ROWS):
                pltpu.async_copy(
                    table_hbm.at[idx_v.at[b, j]],
                    rows_v.at[b, pl.ds(j * GATHER, GATHER)],
                    gsem,
                )

        def wait_gathers(b):
            for j in range(IDX_ROWS):
                pltpu.make_async_copy(
                    table_hbm.at[idx_v.at[b, j]],
                    rows_v.at[b, pl.ds(j * GATHER, GATHER)],
                    gsem,
                ).wait()

        def start_out(c, b):
            pltpu.async_copy(rows_v.at[b], out_hbm.at[out_rows_of(c)], osems[b])

        def wait_out(c, b):
            pltpu.make_async_copy(rows_v.at[b], out_hbm.at[out_rows_of(c)], osems[b]).wait()

        def start_idx(c, b):
            pltpu.async_copy(idx_hbm.at[idx_rows_of(c)], idx_v.at[b], isem)

        def wait_idx(c, b):
            pltpu.make_async_copy(idx_hbm.at[idx_rows_of(c)], idx_v.at[b], isem).wait()

        pltpu.sync_copy(idx_hbm.at[idx_rows_of(0)], idx_v.at[0])
        fire_gathers(0)
        start_idx(1, 1)

        def step(c, b):
            wait_gathers(b)
            start_out(c, b)

            @pl.when(c + 1 < n_chunks)
            def _():
                wait_idx(c + 1, 1 - b)

                @pl.when(c >= 1)
                def _():
                    wait_out(c - 1, 1 - b)

                fire_gathers(1 - b)

            @pl.when(c + 2 < n_chunks)
            def _():
                start_idx(c + 2, b)

        def outer(h, carry):
            step(2 * h, 0)
            step(2 * h + 1, 1)
            return carry

        lax.fori_loop(0, n_outer, outer, 0)
        wait_out(n_chunks - 2, 0)
        wait_out(n_chunks - 1, 1)

    return gather_kernel


def kernel(x, embeddings):
    b, t = x.shape
    total = b * t
    idx2d = x.astype(jnp.int32).reshape(total // GATHER, GATHER)
    out = _build(total)(embeddings, idx2d)
    return out.reshape(b, t, embeddings.shape[1])


# speedup vs baseline: 16.5193x; 1.6920x over previous
import functools

import jax
import jax.numpy as jnp
from jax import lax
from jax.experimental import pallas as pl
from jax.experimental.pallas import tpu as pltpu
from jax.experimental.pallas import tpu_sc as plsc

D = 128
NW = 32
GATHER = 128
IDX_ROWS = 1
CHUNK = IDX_ROWS * GATHER
TABLE_ROWS = 8192


def _build(total_rows: int):
    per_w = total_rows // NW
    n_chunks = per_w // CHUNK
    n_outer = n_chunks // 2
    mesh = plsc.VectorSubcoreMesh(core_axis_name="c", subcore_axis_name="s")

    @functools.partial(
        pl.kernel,
        mesh=mesh,
        out_type=jax.ShapeDtypeStruct((total_rows, D), jnp.float32),
        scratch_types=[
            pltpu.VMEM((2, IDX_ROWS, GATHER), jnp.int32),
            pltpu.VMEM((2, CHUNK, D), jnp.float32),
            pltpu.VMEM_SHARED((TABLE_ROWS, D), jnp.float32),
            pltpu.SemaphoreType.DMA,
            pltpu.SemaphoreType.DMA,
            pltpu.SemaphoreType.DMA,
            pltpu.SemaphoreType.DMA,
        ],
    )
    def gather_kernel(table_hbm, idx_hbm, out_hbm, idx_v, rows_v, table_sh,
                      isem, gsem, os0, os1):
        wid = lax.axis_index("s") * 2 + lax.axis_index("c")
        base_irow = wid * (per_w // GATHER)
        osems = (os0, os1)

        def idx_rows_of(c):
            return pl.ds(base_irow + c * IDX_ROWS, IDX_ROWS)

        def out_rows_of(c):
            return pl.ds((base_irow + c * IDX_ROWS) * GATHER, CHUNK)

        def fire_gathers(b):
            for j in range(IDX_ROWS):
                pltpu.async_copy(
                    table_sh.at[idx_v.at[b, j]],
                    rows_v.at[b, pl.ds(j * GATHER, GATHER)],
                    gsem,
                )

        def wait_gathers(b):
            for j in range(IDX_ROWS):
                pltpu.make_async_copy(
                    table_sh.at[idx_v.at[b, j]],
                    rows_v.at[b, pl.ds(j * GATHER, GATHER)],
                    gsem,
                ).wait()

        def start_out(c, b):
            pltpu.async_copy(rows_v.at[b], out_hbm.at[out_rows_of(c)], osems[b])

        def wait_out(c, b):
            pltpu.make_async_copy(rows_v.at[b], out_hbm.at[out_rows_of(c)], osems[b]).wait()

        def start_idx(c, b):
            pltpu.async_copy(idx_hbm.at[idx_rows_of(c)], idx_v.at[b], isem)

        def wait_idx(c, b):
            pltpu.make_async_copy(idx_hbm.at[idx_rows_of(c)], idx_v.at[b], isem).wait()

        @pl.when(lax.axis_index("s") == 0)
        def _():
            pltpu.sync_copy(table_hbm, table_sh)

        plsc.subcore_barrier()

        pltpu.sync_copy(idx_hbm.at[idx_rows_of(0)], idx_v.at[0])
        fire_gathers(0)
        start_idx(1, 1)

        def step(c, b):
            wait_gathers(b)
            start_out(c, b)

            @pl.when(c + 1 < n_chunks)
            def _():
                wait_idx(c + 1, 1 - b)

                @pl.when(c >= 1)
                def _():
                    wait_out(c - 1, 1 - b)

                fire_gathers(1 - b)

            @pl.when(c + 2 < n_chunks)
            def _():
                start_idx(c + 2, b)

        def outer(h, carry):
            step(2 * h, 0)
            step(2 * h + 1, 1)
            return carry

        lax.fori_loop(0, n_outer, outer, 0)
        wait_out(n_chunks - 2, 0)
        wait_out(n_chunks - 1, 1)

    return gather_kernel


def kernel(x, embeddings):
    b, t = x.shape
    total = b * t
    idx2d = x.astype(jnp.int32).reshape(total // GATHER, GATHER)
    out = _build(total)(embeddings, idx2d)
    return out.reshape(b, t, embeddings.shape[1])
